# speedup vs baseline: 2.3167x; 2.3167x over previous
import functools

import jax
import jax.numpy as jnp
from jax import lax
from jax.experimental import pallas as pl
from jax.experimental.pallas import tpu as pltpu
from jax.experimental.pallas import tpu_sc as plsc


def _tables_body(src_ref, dst_ref, w_ref, b_ref, out_ref):
    pid = pl.program_id(0)
    x = jnp.where(pid == 0, src_ref[...], dst_ref[...])
    w = w_ref[0]
    y = lax.dot_general(x, w, (((1,), (1,)), ((), ())),
                        preferred_element_type=jnp.float32)
    out_ref[0] = y + b_ref[0]


def _node_tables(src_feat, dst_feat, Wsd, bsd, N, SD, H):
    return pl.pallas_call(
        _tables_body,
        grid=(2,),
        in_specs=[
            pl.BlockSpec((N, SD), lambda i: (0, 0)),
            pl.BlockSpec((N, SD), lambda i: (0, 0)),
            pl.BlockSpec((1, H, SD), lambda i: (i, 0, 0)),
            pl.BlockSpec((1, 1, H), lambda i: (i, 0, 0)),
        ],
        out_specs=pl.BlockSpec((1, N, H), lambda i: (i, 0, 0)),
        out_shape=jax.ShapeDtypeStruct((2, N, H), jnp.float32),
    )(src_feat, dst_feat, Wsd, bsd)


_NC = 2
_NS = 16
_NW = _NC * _NS


def _make_sc_gather(total, H, chunk):
    per_w = total // _NW
    n_ch = per_w // chunk
    assert per_w % chunk == 0 and total % _NW == 0 and chunk % 8 == 0
    mesh = plsc.VectorSubcoreMesh(core_axis_name="c", subcore_axis_name="s")

    @functools.partial(
        pl.kernel,
        mesh=mesh,
        out_type=jax.ShapeDtypeStruct((total, H), jnp.float32),
        scratch_types=[
            pltpu.VMEM((chunk,), jnp.int32),
            pltpu.VMEM((chunk, H), jnp.float32),
            pltpu.SemaphoreType.DMA,
        ],
    )
    def gather_kernel(table_hbm, idx_hbm, out_hbm, idx_v, rows_v, sem):
        wid = lax.axis_index("s") * _NC + lax.axis_index("c")
        base = wid * per_w

        @pl.loop(0, n_ch)
        def _(i):
            off = base + i * chunk
            pltpu.sync_copy(idx_hbm.at[pl.ds(off, chunk)], idx_v)
            pltpu.async_copy(table_hbm.at[idx_v], rows_v, sem).wait()
            pltpu.sync_copy(rows_v, out_hbm.at[pl.ds(off, chunk)])

    return gather_kernel


def _edge_body(e_ref, gs_ref, gd_ref, wet_ref, w1t_ref, b1_ref, gam_ref,
               bet_ref, o_ref):
    h = lax.dot_general(e_ref[...], wet_ref[...], (((1,), (0,)), ((), ())),
                        preferred_element_type=jnp.float32)
    h = h + gs_ref[0] + gd_ref[0]
    h = h * jax.nn.sigmoid(h)
    h2 = lax.dot_general(h, w1t_ref[...], (((1,), (0,)), ((), ())),
                         preferred_element_type=jnp.float32)
    h2 = h2 + b1_ref[...]
    mu = jnp.mean(h2, axis=-1, keepdims=True)
    d = h2 - mu
    var = jnp.mean(d * d, axis=-1, keepdims=True)
    o_ref[...] = d * lax.rsqrt(var + 1e-5) * gam_ref[...] + bet_ref[...]


def _edge_pass(efeat, Gr, WeT, W1T, b1, gamma, beta, E, EF, H, OUT, BE):
    return pl.pallas_call(
        _edge_body,
        grid=(E // BE,),
        in_specs=[
            pl.BlockSpec((BE, EF), lambda i: (i, 0)),
            pl.BlockSpec((1, BE, H), lambda i: (0, i, 0)),
            pl.BlockSpec((1, BE, H), lambda i: (1, i, 0)),
            pl.BlockSpec((EF, H), lambda i: (0, 0)),
            pl.BlockSpec((H, OUT), lambda i: (0, 0)),
            pl.BlockSpec((1, OUT), lambda i: (0, 0)),
            pl.BlockSpec((1, OUT), lambda i: (0, 0)),
            pl.BlockSpec((1, OUT), lambda i: (0, 0)),
        ],
        out_specs=pl.BlockSpec((BE, OUT), lambda i: (i, 0)),
        out_shape=jax.ShapeDtypeStruct((E, OUT), jnp.float32),
    )(efeat, Gr, Gr, WeT, W1T, b1, gamma, beta)


def kernel(efeat, src_feat, dst_feat, edge_index, W_e, W_s, W_d, b0, W1, b1,
           gamma, beta):
    E, EF = efeat.shape
    N, SD = src_feat.shape
    H = W_e.shape[0]
    OUT = W1.shape[0]

    Wsd = jnp.stack([W_s, W_d])
    bsd = jnp.stack([jnp.zeros_like(b0), b0]).reshape(2, 1, H)
    T = _node_tables(src_feat, dst_feat, Wsd, bsd, N, SD, H)
    T2 = T.reshape(2 * N, H)

    J = (edge_index + jnp.array([[0], [N]], jnp.int32)).reshape(-1)

    G = _make_sc_gather(2 * E, H, chunk=80)(T2, J)
    Gr = G.reshape(2, E, H)

    return _edge_pass(efeat, Gr, W_e.T, W1.T, b1.reshape(1, OUT),
                      gamma.reshape(1, OUT), beta.reshape(1, OUT),
                      E, EF, H, OUT, BE=2000)
